# baseline (speedup 1.0000x reference)
import functools

import jax
import jax.numpy as jnp
from jax import lax
from jax.experimental import pallas as pl
from jax.experimental.pallas import tpu as pltpu
from jax.experimental.pallas import tpu_sc as plsc

_B = 128
_R = 16
_T = 77
_TPR = 4
_NCTX = 16
_NSTAIN = 4
_D = 512
_VIS = 512
_VOCAB = 49408
_PURE = _NCTX + _NSTAIN + _TPR
_EOT = 1 + _PURE + 1

_NC = 2
_NS = 16
_NW = _NC * _NS
_NIDX = _B * _NSTAIN
_IPW = _NIDX // _NW

_NBUF = 4


def _sc_gather(table, idx):
    mesh = plsc.VectorSubcoreMesh(core_axis_name="c", subcore_axis_name="s")

    @functools.partial(
        pl.kernel,
        mesh=mesh,
        out_type=jax.ShapeDtypeStruct((_NIDX, _D), jnp.float32),
        scratch_types=[
            pltpu.VMEM((_IPW,), jnp.int32),
            pltpu.VMEM((_IPW, _D), jnp.float32),
            pltpu.SemaphoreType.DMA,
        ],
    )
    def gather_kernel(table_hbm, idx_hbm, out_hbm, idx_v, rows_v, sem):
        wid = lax.axis_index("s") * _NC + lax.axis_index("c")
        base = wid * _IPW
        pltpu.sync_copy(idx_hbm.at[pl.ds(base, _IPW)], idx_v)
        pltpu.async_copy(table_hbm.at[idx_v], rows_v, sem).wait()
        pltpu.sync_copy(rows_v, out_hbm.at[pl.ds(base, _IPW)])

    return gather_kernel(table, idx)


def _assemble_body(im_ref, ime_ref, w1_ref, b1_ref, w2_ref, b2_ref,
                   we1_ref, be1_ref, we2_ref, be2_ref,
                   ctx_ref, rank_ref, stain_ref, spec_ref,
                   sent_hbm, tok_ref, vbuf, sems):
    b = pl.program_id(0)
    slot = lax.rem(b, _NBUF)

    @pl.when(b >= _NBUF)
    def _drain():
        pltpu.make_async_copy(vbuf.at[slot], sent_hbm.at[b], sems.at[slot]).wait()

    x = im_ref[:, 0, :]
    h = jnp.maximum(
        jnp.dot(x, w1_ref[...], preferred_element_type=jnp.float32) + b1_ref[...],
        0.0)
    bias_g = jnp.dot(h, w2_ref[...], preferred_element_type=jnp.float32) + b2_ref[...]
    xe = ime_ref[:, 0, :]
    he = jnp.maximum(
        jnp.dot(xe, we1_ref[...], preferred_element_type=jnp.float32) + be1_ref[...],
        0.0)
    bias_e = jnp.dot(he, we2_ref[...], preferred_element_type=jnp.float32) + be2_ref[...]

    null_e = spec_ref[0:1, :]
    fs_e = spec_ref[1:2, :]
    sot_e = spec_ref[2:3, :]
    eot_e = spec_ref[3:4, :]

    ctx_g = ctx_ref[0:_NCTX // 2, :] + bias_g
    ctx_e = ctx_ref[_NCTX // 2:, :] + bias_e
    stain = stain_ref[0]

    v = vbuf.at[slot]
    v[:, 0:1, :] = jnp.broadcast_to(sot_e[None], (_R, 1, _D))
    v[:, 1:9, :] = jnp.broadcast_to(ctx_g[None], (_R, 8, _D))
    v[:, 9:13, :] = jnp.broadcast_to(stain[None], (_R, _NSTAIN, _D))
    v[:, 13:21, :] = jnp.broadcast_to(ctx_e[None], (_R, 8, _D))
    v[:, 21:25, :] = rank_ref[...]
    v[:, 25:26, :] = jnp.broadcast_to(fs_e[None], (_R, 1, _D))
    v[:, 26:27, :] = jnp.broadcast_to(eot_e[None], (_R, 1, _D))
    v[:, 27:_T, :] = jnp.broadcast_to(null_e[None], (_R, _T - 27, _D))

    pltpu.make_async_copy(vbuf.at[slot], sent_hbm.at[b], sems.at[slot]).start()

    t = lax.broadcasted_iota(jnp.int32, (_R, _T), 1)
    tok_ref[0] = jnp.where(
        t == 0, 49406,
        jnp.where(t <= _PURE, 1,
                  jnp.where(t == _EOT - 1, 269,
                            jnp.where(t == _EOT, 49407, 0))))

    @pl.when(b == _B - 1)
    def _final():
        for k in range(_NBUF):
            pltpu.make_async_copy(vbuf.at[k], sent_hbm.at[b], sems.at[k]).wait()


def _assemble(im_features, im_features_edge, W1, b1, W2, b2,
              We1, be1, We2, be2, context_embeds, rank_embeds,
              stain_emb, specials):
    h = W1.shape[1]
    grid = (_B,)
    full = lambda shape: pl.BlockSpec(shape, lambda b: (0,) * len(shape))
    return pl.pallas_call(
        _assemble_body,
        grid=grid,
        in_specs=[
            pl.BlockSpec((1, 1, _VIS), lambda b: (b, 0, 0)),
            pl.BlockSpec((1, 1, _VIS), lambda b: (b, 0, 0)),
            full((_VIS, h)),
            full((1, h)),
            full((h, _D)),
            full((1, _D)),
            full((_VIS, h)),
            full((1, h)),
            full((h, _D)),
            full((1, _D)),
            full((_NCTX, _D)),
            full((_R, _TPR, _D)),
            pl.BlockSpec((1, _NSTAIN, _D), lambda b: (b, 0, 0)),
            full((4, _D)),
        ],
        out_specs=[
            pl.BlockSpec(memory_space=pl.ANY),
            pl.BlockSpec((1, _R, _T), lambda b: (b, 0, 0)),
        ],
        out_shape=[
            jax.ShapeDtypeStruct((_B, _R, _T, _D), jnp.float32),
            jax.ShapeDtypeStruct((_B, _R, _T), jnp.int32),
        ],
        scratch_shapes=[
            pltpu.VMEM((_NBUF, _R, _T, _D), jnp.float32),
            pltpu.SemaphoreType.DMA((_NBUF,)),
        ],
        compiler_params=pltpu.CompilerParams(
            dimension_semantics=("arbitrary",)),
    )(im_features[:, None, :], im_features_edge[:, None, :],
      W1, b1, W2, b2, We1, be1, We2, be2,
      context_embeds, rank_embeds, stain_emb, specials)


def kernel(im_features, im_features_edge, stains, context_embeds, rank_embeds,
           token_embedding, W1, b1, W2, b2, We1, be1, We2, be2):
    idx = stains.reshape(-1).astype(jnp.int32)
    stain_rows = _sc_gather(token_embedding, idx)
    stain_emb = stain_rows.reshape(_B, _NSTAIN, _D)
    specials = jnp.concatenate([
        token_embedding[0:1],
        token_embedding[269:270],
        token_embedding[49406:49407],
        token_embedding[49407:49408],
    ], axis=0)
    sent, tokens = _assemble(
        im_features, im_features_edge, W1, b1.reshape(1, -1), W2,
        b2.reshape(1, -1), We1, be1.reshape(1, -1), We2, be2.reshape(1, -1),
        context_embeds, rank_embeds, stain_emb, specials)
    return (sent, tokens)


# speedup vs baseline: 2.2832x; 2.2832x over previous
import functools

import jax
import jax.numpy as jnp
from jax import lax
from jax.experimental import pallas as pl
from jax.experimental.pallas import tpu as pltpu
from jax.experimental.pallas import tpu_sc as plsc

_B = 128
_R = 16
_T = 77
_TPR = 4
_NCTX = 16
_NSTAIN = 4
_D = 512
_VIS = 512
_VOCAB = 49408
_PURE = _NCTX + _NSTAIN + _TPR
_EOT = 1 + _PURE + 1

_NC = 2
_NS = 16
_NW = _NC * _NS
_NIDX = _B * _NSTAIN
_IPW = _NIDX // _NW


def _sc_gather(table, idx):
    mesh = plsc.VectorSubcoreMesh(core_axis_name="c", subcore_axis_name="s")

    @functools.partial(
        pl.kernel,
        mesh=mesh,
        out_type=jax.ShapeDtypeStruct((_NIDX, _D), jnp.float32),
        scratch_types=[
            pltpu.VMEM((_IPW,), jnp.int32),
            pltpu.VMEM((_IPW, _D), jnp.float32),
            pltpu.SemaphoreType.DMA,
        ],
    )
    def gather_kernel(table_hbm, idx_hbm, out_hbm, idx_v, rows_v, sem):
        wid = lax.axis_index("s") * _NC + lax.axis_index("c")
        base = wid * _IPW
        pltpu.sync_copy(idx_hbm.at[pl.ds(base, _IPW)], idx_v)
        pltpu.async_copy(table_hbm.at[idx_v], rows_v, sem).wait()
        pltpu.sync_copy(rows_v, out_hbm.at[pl.ds(base, _IPW)])

    return gather_kernel(table, idx)


def _assemble_body(im_ref, ime_ref, w1_ref, b1_ref, w2_ref, b2_ref,
                   we1_ref, be1_ref, we2_ref, be2_ref,
                   ctx_ref, rank_ref, stain_ref, spec_ref,
                   sent_ref, tok_ref):
    b = pl.program_id(0)

    x = im_ref[:, 0, :]
    h = jnp.maximum(
        jnp.dot(x, w1_ref[...], preferred_element_type=jnp.float32) + b1_ref[...],
        0.0)
    bias_g = jnp.dot(h, w2_ref[...], preferred_element_type=jnp.float32) + b2_ref[...]
    xe = ime_ref[:, 0, :]
    he = jnp.maximum(
        jnp.dot(xe, we1_ref[...], preferred_element_type=jnp.float32) + be1_ref[...],
        0.0)
    bias_e = jnp.dot(he, we2_ref[...], preferred_element_type=jnp.float32) + be2_ref[...]

    null_e = spec_ref[0:1, :]
    fs_e = spec_ref[1:2, :]
    sot_e = spec_ref[2:3, :]
    eot_e = spec_ref[3:4, :]

    ctx_g = ctx_ref[0:_NCTX // 2, :] + bias_g
    ctx_e = ctx_ref[_NCTX // 2:, :] + bias_e
    stain = stain_ref[0]

    sent_ref[0, 0:1] = jnp.broadcast_to(sot_e[:, None, :], (1, _R, _D))
    sent_ref[0, 1:9] = jnp.broadcast_to(ctx_g[:, None, :], (8, _R, _D))
    sent_ref[0, 9:13] = jnp.broadcast_to(stain[:, None, :], (_NSTAIN, _R, _D))
    sent_ref[0, 13:21] = jnp.broadcast_to(ctx_e[:, None, :], (8, _R, _D))
    sent_ref[0, 21:25] = rank_ref[...]
    sent_ref[0, 25:26] = jnp.broadcast_to(fs_e[:, None, :], (1, _R, _D))
    sent_ref[0, 26:27] = jnp.broadcast_to(eot_e[:, None, :], (1, _R, _D))
    sent_ref[0, 27:_T] = jnp.broadcast_to(null_e[:, None, :], (_T - 27, _R, _D))

    @pl.when(b == 0)
    def _tokens():
        t = lax.broadcasted_iota(jnp.int32, (_T, _R, _B), 0)
        tok_ref[...] = jnp.where(
            t == 0, 49406,
            jnp.where(t <= _PURE, 1,
                      jnp.where(t == _EOT - 1, 269,
                                jnp.where(t == _EOT, 49407, 0))))


def _assemble(im_features, im_features_edge, W1, b1, W2, b2,
              We1, be1, We2, be2, context_embeds, rank_embeds,
              stain_emb, specials):
    h = W1.shape[1]
    grid = (_B,)
    full = lambda shape: pl.BlockSpec(shape, lambda b: (0,) * len(shape))
    return pl.pallas_call(
        _assemble_body,
        grid=grid,
        in_specs=[
            pl.BlockSpec((1, 1, _VIS), lambda b: (b, 0, 0)),
            pl.BlockSpec((1, 1, _VIS), lambda b: (b, 0, 0)),
            full((_VIS, h)),
            full((1, h)),
            full((h, _D)),
            full((1, _D)),
            full((_VIS, h)),
            full((1, h)),
            full((h, _D)),
            full((1, _D)),
            full((_NCTX, _D)),
            full((_TPR, _R, _D)),
            pl.BlockSpec((1, _NSTAIN, _D), lambda b: (b, 0, 0)),
            full((4, _D)),
        ],
        out_specs=[
            pl.BlockSpec((1, _T, _R, _D), lambda b: (b, 0, 0, 0)),
            full((_T, _R, _B)),
        ],
        out_shape=[
            jax.ShapeDtypeStruct((_B, _T, _R, _D), jnp.float32),
            jax.ShapeDtypeStruct((_T, _R, _B), jnp.int32),
        ],
        compiler_params=pltpu.CompilerParams(
            dimension_semantics=("arbitrary",)),
    )(im_features[:, None, :], im_features_edge[:, None, :],
      W1, b1, W2, b2, We1, be1, We2, be2,
      context_embeds, rank_embeds, stain_emb, specials)


def kernel(im_features, im_features_edge, stains, context_embeds, rank_embeds,
           token_embedding, W1, b1, W2, b2, We1, be1, We2, be2):
    idx = stains.reshape(-1).astype(jnp.int32)
    stain_rows = _sc_gather(token_embedding, idx)
    stain_emb = stain_rows.reshape(_B, _NSTAIN, _D)
    specials = jnp.concatenate([
        token_embedding[0:1],
        token_embedding[269:270],
        token_embedding[49406:49407],
        token_embedding[49407:49408],
    ], axis=0)
    rank_t = jnp.transpose(rank_embeds, (1, 0, 2))
    sent_p, tok_p = _assemble(
        im_features, im_features_edge, W1, b1.reshape(1, -1), W2,
        b2.reshape(1, -1), We1, be1.reshape(1, -1), We2, be2.reshape(1, -1),
        context_embeds, rank_t, stain_emb, specials)
    sent = jnp.transpose(sent_p, (0, 2, 1, 3))
    tokens = jnp.transpose(tok_p, (2, 1, 0))
    return (sent, tokens)


# speedup vs baseline: 2.9939x; 1.3113x over previous
import functools

import jax
import jax.numpy as jnp
from jax import lax
from jax.experimental import pallas as pl
from jax.experimental.pallas import tpu as pltpu
from jax.experimental.pallas import tpu_sc as plsc

_B = 128
_R = 16
_T = 77
_TPR = 4
_NCTX = 16
_NSTAIN = 4
_D = 512
_VIS = 512
_VOCAB = 49408
_PURE = _NCTX + _NSTAIN + _TPR
_EOT = 1 + _PURE + 1

_NC = 2
_NS = 16
_NW = _NC * _NS
_NIDX = _B * _NSTAIN
_IPW = _NIDX // _NW

_NB = 2


def _sc_gather(table, idx):
    mesh = plsc.VectorSubcoreMesh(core_axis_name="c", subcore_axis_name="s")

    @functools.partial(
        pl.kernel,
        mesh=mesh,
        out_type=jax.ShapeDtypeStruct((_NIDX, _D), jnp.float32),
        scratch_types=[
            pltpu.VMEM((_IPW,), jnp.int32),
            pltpu.VMEM((_IPW, _D), jnp.float32),
            pltpu.SemaphoreType.DMA,
        ],
    )
    def gather_kernel(table_hbm, idx_hbm, out_hbm, idx_v, rows_v, sem):
        wid = lax.axis_index("s") * _NC + lax.axis_index("c")
        base = wid * _IPW
        pltpu.sync_copy(idx_hbm.at[pl.ds(base, _IPW)], idx_v)
        pltpu.async_copy(table_hbm.at[idx_v], rows_v, sem).wait()
        pltpu.sync_copy(rows_v, out_hbm.at[pl.ds(base, _IPW)])

    return gather_kernel(table, idx)


def _assemble_body(im_ref, ime_ref, w1_ref, b1_ref, w2_ref, b2_ref,
                   we1_ref, be1_ref, we2_ref, be2_ref,
                   ctx_ref, rank_ref, stain_ref, spec_ref,
                   sent_ref, tok_ref):
    b = pl.program_id(0)

    x = im_ref[:, 0, :]
    h = jnp.maximum(
        jnp.dot(x, w1_ref[...], preferred_element_type=jnp.float32) + b1_ref[...],
        0.0)
    bias_g = jnp.dot(h, w2_ref[...], preferred_element_type=jnp.float32) + b2_ref[...]
    xe = ime_ref[:, 0, :]
    he = jnp.maximum(
        jnp.dot(xe, we1_ref[...], preferred_element_type=jnp.float32) + be1_ref[...],
        0.0)
    bias_e = jnp.dot(he, we2_ref[...], preferred_element_type=jnp.float32) + be2_ref[...]

    null_e = spec_ref[0:1, :]
    fs_e = spec_ref[1:2, :]
    sot_e = spec_ref[2:3, :]
    eot_e = spec_ref[3:4, :]

    for i in range(_NB):
        ctx_g = ctx_ref[0:_NCTX // 2, :] + bias_g[i:i + 1, :]
        ctx_e = ctx_ref[_NCTX // 2:, :] + bias_e[i:i + 1, :]
        stain = stain_ref[i]
        sent_ref[i, 0:1] = jnp.broadcast_to(sot_e[:, None, :], (1, _R, _D))
        sent_ref[i, 1:9] = jnp.broadcast_to(ctx_g[:, None, :], (8, _R, _D))
        sent_ref[i, 9:13] = jnp.broadcast_to(stain[:, None, :], (_NSTAIN, _R, _D))
        sent_ref[i, 13:21] = jnp.broadcast_to(ctx_e[:, None, :], (8, _R, _D))
        sent_ref[i, 21:25] = rank_ref[...]
        sent_ref[i, 25:26] = jnp.broadcast_to(fs_e[:, None, :], (1, _R, _D))
        sent_ref[i, 26:27] = jnp.broadcast_to(eot_e[:, None, :], (1, _R, _D))
        sent_ref[i, 27:_T] = jnp.broadcast_to(null_e[:, None, :], (_T - 27, _R, _D))

    @pl.when(b == 0)
    def _tokens():
        t = lax.broadcasted_iota(jnp.int32, (_T, _R, _B), 0)
        tok_ref[...] = jnp.where(
            t == 0, 49406,
            jnp.where(t <= _PURE, 1,
                      jnp.where(t == _EOT - 1, 269,
                                jnp.where(t == _EOT, 49407, 0))))


def _assemble(im_features, im_features_edge, W1, b1, W2, b2,
              We1, be1, We2, be2, context_embeds, rank_embeds,
              stain_emb, specials):
    h = W1.shape[1]
    grid = (_B // _NB,)
    full = lambda shape: pl.BlockSpec(shape, lambda b: (0,) * len(shape))
    return pl.pallas_call(
        _assemble_body,
        grid=grid,
        in_specs=[
            pl.BlockSpec((_NB, 1, _VIS), lambda b: (b, 0, 0)),
            pl.BlockSpec((_NB, 1, _VIS), lambda b: (b, 0, 0)),
            full((_VIS, h)),
            full((1, h)),
            full((h, _D)),
            full((1, _D)),
            full((_VIS, h)),
            full((1, h)),
            full((h, _D)),
            full((1, _D)),
            full((_NCTX, _D)),
            full((_TPR, _R, _D)),
            pl.BlockSpec((_NB, _NSTAIN, _D), lambda b: (b, 0, 0)),
            full((4, _D)),
        ],
        out_specs=[
            pl.BlockSpec((_NB, _T, _R, _D), lambda b: (b, 0, 0, 0)),
            full((_T, _R, _B)),
        ],
        out_shape=[
            jax.ShapeDtypeStruct((_B, _T, _R, _D), jnp.float32),
            jax.ShapeDtypeStruct((_T, _R, _B), jnp.int32),
        ],
        compiler_params=pltpu.CompilerParams(
            dimension_semantics=("arbitrary",)),
    )(im_features[:, None, :], im_features_edge[:, None, :],
      W1, b1, W2, b2, We1, be1, We2, be2,
      context_embeds, rank_embeds, stain_emb, specials)


def kernel(im_features, im_features_edge, stains, context_embeds, rank_embeds,
           token_embedding, W1, b1, W2, b2, We1, be1, We2, be2):
    idx = stains.reshape(-1).astype(jnp.int32)
    stain_rows = _sc_gather(token_embedding, idx)
    stain_emb = stain_rows.reshape(_B, _NSTAIN, _D)
    specials = jnp.concatenate([
        token_embedding[0:1],
        token_embedding[269:270],
        token_embedding[49406:49407],
        token_embedding[49407:49408],
    ], axis=0)
    rank_t = jnp.transpose(rank_embeds, (1, 0, 2))
    sent_p, tok_p = _assemble(
        im_features, im_features_edge, W1, b1.reshape(1, -1), W2,
        b2.reshape(1, -1), We1, be1.reshape(1, -1), We2, be2.reshape(1, -1),
        context_embeds, rank_t, stain_emb, specials)
    sent = jnp.transpose(sent_p, (0, 2, 1, 3))
    tokens = jnp.transpose(tok_p, (2, 1, 0))
    return (sent, tokens)
